# Initial kernel scaffold; baseline (speedup 1.0000x reference)
import jax
import jax.numpy as jnp
from jax.experimental import pallas as pl


def kernel(x, edge_index, W_gat, att_src, att_dst, b_gat, Wa, ba, W1, b1, W2, b2):
    raise NotImplementedError("write your pallas kernel here")


# speedup vs baseline: 5.6868x; 5.6868x over previous
import dataclasses
import functools

import jax
import jax.numpy as jnp
from jax import lax
from jax.experimental import pallas as pl
from jax.experimental.pallas import tpu as pltpu
from jax.experimental.pallas import tpu_sc as plsc

N = 10000
E = 160000
D_IN = 512
H = 2
D_OUT = 512
D = H * D_OUT
E_REAL = E + N
NS = 16
IDX_ROW = 128
EP = 172032
PW = EP // NS
NR = PW // IDX_ROW
NG2 = PW // 16 + 16
R = 1536
NPASS = 4
N_PAD = 2 * R * NPASS
ROW_T = 400
COL_T = 1024


def _sc_params():
    cp = pltpu.CompilerParams()
    fields = pltpu.CompilerParams.__dataclass_fields__
    if "needs_layout_passes" in fields:
        cp = dataclasses.replace(cp, needs_layout_passes=False)
    if "use_tc_tiling_on_sc" in fields:
        cp = dataclasses.replace(cp, use_tc_tiling_on_sc=False)
    return cp


def _stage1_body(x_ref, w_ref, atts_ref, attd_ref, xh_ref, asrc_ref, adst_ref,
                 m_ref):
    i = pl.program_id(0)
    xh = jnp.dot(x_ref[...], w_ref[...], preferred_element_type=jnp.float32)
    xh_ref[...] = xh
    rs = xh * atts_ref[...]
    rd = xh * attd_ref[...]
    as0 = jnp.sum(rs[:, :D_OUT], axis=1)
    as1 = jnp.sum(rs[:, D_OUT:], axis=1)
    ad0 = jnp.sum(rd[:, :D_OUT], axis=1)
    ad1 = jnp.sum(rd[:, D_OUT:], axis=1)
    asrc_ref[...] = jnp.stack([as0, as1], axis=1)
    adst_ref[...] = jnp.stack([ad0, ad1], axis=1)
    mt = jnp.stack([jnp.max(as0), jnp.max(as1)])[None, :]

    @pl.when(i == 0)
    def _():
        m_ref[...] = mt

    @pl.when(i > 0)
    def _():
        m_ref[...] = jnp.maximum(m_ref[...], mt)


def _stage1(x, w_gat, atts, attd):
    return pl.pallas_call(
        _stage1_body,
        grid=(N // ROW_T,),
        in_specs=[
            pl.BlockSpec((ROW_T, D_IN), lambda i: (i, 0)),
            pl.BlockSpec((D_IN, D), lambda i: (0, 0)),
            pl.BlockSpec((1, D), lambda i: (0, 0)),
            pl.BlockSpec((1, D), lambda i: (0, 0)),
        ],
        out_specs=[
            pl.BlockSpec((ROW_T, D), lambda i: (i, 0)),
            pl.BlockSpec((ROW_T, 2), lambda i: (i, 0)),
            pl.BlockSpec((ROW_T, 2), lambda i: (i, 0)),
            pl.BlockSpec((1, 2), lambda i: (0, 0)),
        ],
        out_shape=[
            jax.ShapeDtypeStruct((N, D), jnp.float32),
            jax.ShapeDtypeStruct((N, 2), jnp.float32),
            jax.ShapeDtypeStruct((N, 2), jnp.float32),
            jax.ShapeDtypeStruct((1, 2), jnp.float32),
        ],
    )(x, w_gat, atts, attd)


def _attn_body(src_h, dst_h, as0_h, as1_h, ad0_h, ad1_h, m0_h, m1_h,
               cnt_h, srcl_h, rll_h, w0l_h, w1l_h, asum_h,
               src_v, dst_v, as0_v, as1_v, ad0_v, ad1_v,
               m0_v, m1_v, zb_v, wr0_v, wr1_v,
               srcm_v, rlm_v, w0m_v, w1m_v, cnt_v,
               spm0, spm1, sem):
    c = lax.axis_index("c")
    s = lax.axis_index("s")
    pltpu.sync_copy(src_h.at[s], src_v)
    pltpu.sync_copy(dst_h.at[s], dst_v)
    pltpu.sync_copy(as0_h, as0_v)
    pltpu.sync_copy(as1_h, as1_v)
    pltpu.sync_copy(ad0_h, ad0_v)
    pltpu.sync_copy(ad1_h, ad1_v)
    pltpu.sync_copy(m0_h, m0_v)
    pltpu.sync_copy(m1_h, m1_v)

    m0 = m0_v[...]
    m1 = m1_v[...]
    base = s * PW

    def edge_w(r, k):
        off = r * IDX_ROW + k * 16
        s16 = src_v[r, pl.ds(k * 16, 16)]
        d16 = dst_v[r, pl.ds(k * 16, 16)]
        a_s0 = plsc.load_gather(as0_v, [s16])
        a_s1 = plsc.load_gather(as1_v, [s16])
        a_d0 = plsc.load_gather(ad0_v, [d16])
        a_d1 = plsc.load_gather(ad1_v, [d16])
        t0 = a_s0 + a_d0
        t1 = a_s1 + a_d1
        t0 = jnp.where(t0 >= 0.0, t0, 0.2 * t0)
        t1 = jnp.where(t1 >= 0.0, t1, 0.2 * t1)
        u0 = m0 + a_d0
        u1 = m1 + a_d1
        u0 = jnp.where(u0 >= 0.0, u0, 0.2 * u0)
        u1 = jnp.where(u1 >= 0.0, u1, 0.2 * u1)
        w0 = jnp.exp(t0 - u0)
        w1 = jnp.exp(t1 - u1)
        valid = (base + off + lax.iota(jnp.int32, 16)) < E_REAL
        w0 = jnp.where(valid, w0, 0.0)
        w1 = jnp.where(valid, w1, 0.0)
        return s16, d16, w0, w1, valid

    @pl.when(c == 0)
    def _():
        @pl.loop(0, 125)
        def _(i):
            zb_v[pl.ds(i * 16, 16)] = jnp.zeros((16,), jnp.float32)

        @pl.when(s == 0)
        def _():
            for q in range(5):
                pltpu.sync_copy(zb_v, spm0.at[pl.ds(q * 2000, 2000)])

        @pl.when(s == 1)
        def _():
            for q in range(5):
                pltpu.sync_copy(zb_v, spm1.at[pl.ds(q * 2000, 2000)])

        plsc.subcore_barrier()

        @pl.loop(0, NR)
        def _(r):
            for k in range(8):
                _, _, w0, w1, _ = edge_w(r, k)
                wr0_v[pl.ds(k * 16, 16)] = w0
                wr1_v[pl.ds(k * 16, 16)] = w1
            pltpu.async_copy(wr0_v, spm0.at[dst_v.at[r]], sem,
                             add=True).wait()
            pltpu.async_copy(wr1_v, spm1.at[dst_v.at[r]], sem,
                             add=True).wait()

        plsc.subcore_barrier()

        @pl.when(s == 0)
        def _():
            pltpu.sync_copy(spm0, asum_h.at[0, pl.ds(0, N)])

        @pl.when(s == 1)
        def _():
            pltpu.sync_copy(spm1, asum_h.at[1, pl.ds(0, N)])

    cnt_vec = jnp.zeros((16,), jnp.int32)
    lane = lax.iota(jnp.int32, 16)
    for p in range(NPASS):
        cbase = (p * 2 + c) * R

        def comp_body(r, cnt):
            for k in range(8):
                s16, d16, w0, w1, valid = edge_w(r, k)
                rl = d16 - cbase
                m = (rl >= 0) & (rl < R) & valid
                csum = plsc.cumsum(m.astype(jnp.int32))
                tgt = cnt + csum - 1
                row = lax.shift_right_logical(tgt, 4)
                col = lax.bitwise_and(tgt, 15)
                plsc.store_scatter(srcm_v, [row, col], s16, mask=m)
                plsc.store_scatter(rlm_v, [row, col], rl, mask=m)
                plsc.store_scatter(w0m_v, [row, col], w0, mask=m)
                plsc.store_scatter(w1m_v, [row, col], w1, mask=m)
                cnt = cnt + csum[15]
            return cnt

        cnt = lax.fori_loop(0, NR, comp_body, jnp.int32(0))

        tgt = cnt + lax.iota(jnp.int32, 16)
        row = lax.shift_right_logical(tgt, 4)
        col = lax.bitwise_and(tgt, 15)
        zf = jnp.zeros((16,), jnp.float32)
        zi = jnp.zeros((16,), jnp.int32)
        plsc.store_scatter(w0m_v, [row, col], zf)
        plsc.store_scatter(w1m_v, [row, col], zf)
        plsc.store_scatter(rlm_v, [row, col], zi)
        plsc.store_scatter(srcm_v, [row, col], zi)
        cnt_vec = jnp.where(lane == p, cnt, cnt_vec)

        pltpu.sync_copy(srcm_v, srcl_h.at[c, s, p])
        pltpu.sync_copy(rlm_v, rll_h.at[c, s, p])
        pltpu.sync_copy(w0m_v, w0l_h.at[c, s, p])
        pltpu.sync_copy(w1m_v, w1l_h.at[c, s, p])

    cnt_v[...] = cnt_vec
    pltpu.sync_copy(cnt_v, cnt_h.at[c, s])


def _stage_attn(src3, dst3, as0, as1, ad0, ad1, m0, m1):
    mesh = plsc.VectorSubcoreMesh(core_axis_name="c", subcore_axis_name="s")
    kern = pl.kernel(
        _attn_body,
        out_type=[
            jax.ShapeDtypeStruct((2, NS, 16), jnp.int32),
            jax.ShapeDtypeStruct((2, NS, NPASS, NG2, 16), jnp.int32),
            jax.ShapeDtypeStruct((2, NS, NPASS, NG2, 16), jnp.int32),
            jax.ShapeDtypeStruct((2, NS, NPASS, NG2, 16), jnp.float32),
            jax.ShapeDtypeStruct((2, NS, NPASS, NG2, 16), jnp.float32),
            jax.ShapeDtypeStruct((2, N_PAD), jnp.float32),
        ],
        mesh=mesh,
        scratch_types=[
            pltpu.VMEM((NR, IDX_ROW), jnp.int32),
            pltpu.VMEM((NR, IDX_ROW), jnp.int32),
            pltpu.VMEM((N,), jnp.float32),
            pltpu.VMEM((N,), jnp.float32),
            pltpu.VMEM((N,), jnp.float32),
            pltpu.VMEM((N,), jnp.float32),
            pltpu.VMEM((16,), jnp.float32),
            pltpu.VMEM((16,), jnp.float32),
            pltpu.VMEM((2000,), jnp.float32),
            pltpu.VMEM((IDX_ROW,), jnp.float32),
            pltpu.VMEM((IDX_ROW,), jnp.float32),
            pltpu.VMEM((NG2, 16), jnp.int32),
            pltpu.VMEM((NG2, 16), jnp.int32),
            pltpu.VMEM((NG2, 16), jnp.float32),
            pltpu.VMEM((NG2, 16), jnp.float32),
            pltpu.VMEM((16,), jnp.int32),
            pltpu.VMEM_SHARED((N,), jnp.float32),
            pltpu.VMEM_SHARED((N,), jnp.float32),
            pltpu.SemaphoreType.DMA,
        ],
        compiler_params=_sc_params(),
    )
    return kern(src3, dst3, as0, as1, ad0, ad1, m0, m1)


def _agg_body(cnt_h, srcl_h, rll_h, w0l_h, w1l_h, asum_h, xh_h, agg_h,
              srcb_v, rlb_v, w0b_v, w1b_v, asc0_v, asc1_v, rowbuf,
              cnt_v, chunk, sem):
    c = lax.axis_index("c")
    s = lax.axis_index("s")
    pltpu.sync_copy(cnt_h.at[c, s], cnt_v)
    lane = lax.iota(jnp.int32, 16)
    rps = R // NS

    def do_group(j):
        pltpu.async_copy(xh_h.at[srcb_v.at[j]], rowbuf, sem).wait()

        @pl.loop(0, 16)
        def _(e):
            jv = jnp.full((16,), j, jnp.int32)
            ev = jnp.full((16,), e, jnp.int32)
            rlb = plsc.load_gather(rlb_v, [jv, ev])
            a0 = plsc.load_gather(asc0_v, [rlb])
            a1 = plsc.load_gather(asc1_v, [rlb])
            w0 = plsc.load_gather(w0b_v, [jv, ev])
            w1 = plsc.load_gather(w1b_v, [jv, ev])
            s0 = w0 / (a0 + 1e-16)
            s1 = w1 / (a1 + 1e-16)

            @pl.loop(0, D_OUT // 16)
            def _(k):
                sl = pl.ds(k * 16, 16)
                rowbuf[e, sl] = rowbuf[e, sl] * s0

            @pl.loop(D_OUT // 16, D // 16)
            def _(k):
                sl = pl.ds(k * 16, 16)
                rowbuf[e, sl] = rowbuf[e, sl] * s1

        pltpu.async_copy(rowbuf, chunk.at[rlb_v.at[j]], sem, add=True).wait()

    for p in range(NPASS):
        cbase = (p * 2 + c) * R

        @pl.loop(0, 16)
        def _(j):
            @pl.loop(0, D // 16)
            def _(k):
                rowbuf[j, pl.ds(k * 16, 16)] = jnp.zeros((16,), jnp.float32)

        for q in range(rps // 16):
            pltpu.sync_copy(rowbuf, chunk.at[pl.ds(s * rps + q * 16, 16)])

        pltpu.sync_copy(asum_h.at[0, pl.ds(cbase, R)], asc0_v)
        pltpu.sync_copy(asum_h.at[1, pl.ds(cbase, R)], asc1_v)
        plsc.subcore_barrier()

        cnt = jnp.max(jnp.where(lane == p, cnt_v[...], jnp.int32(0)))
        nblk = lax.shift_right_logical(cnt, 8)
        nrem = lax.shift_right_logical(cnt - (nblk << 8) + 15, 4)

        def stage_block(b):
            pltpu.sync_copy(srcl_h.at[c, s, p, pl.ds(b * 16, 16)], srcb_v)
            pltpu.sync_copy(rll_h.at[c, s, p, pl.ds(b * 16, 16)], rlb_v)
            pltpu.sync_copy(w0l_h.at[c, s, p, pl.ds(b * 16, 16)], w0b_v)
            pltpu.sync_copy(w1l_h.at[c, s, p, pl.ds(b * 16, 16)], w1b_v)

        @pl.loop(0, nblk)
        def _(b):
            stage_block(b)

            @pl.loop(0, 16)
            def _(j):
                do_group(j)

        stage_block(nblk)

        @pl.loop(0, nrem)
        def _(j):
            do_group(j)

        plsc.subcore_barrier()

        for q in range(rps // 16):
            r0 = s * rps + q * 16
            pltpu.sync_copy(chunk.at[pl.ds(r0, 16)],
                            agg_h.at[pl.ds(cbase + r0, 16)])
        plsc.subcore_barrier()


def _stage_agg(cnts, srcl, rll, w0l, w1l, asum, xh):
    mesh = plsc.VectorSubcoreMesh(core_axis_name="c", subcore_axis_name="s")
    kern = pl.kernel(
        _agg_body,
        out_type=jax.ShapeDtypeStruct((N_PAD, D), jnp.float32),
        mesh=mesh,
        scratch_types=[
            pltpu.VMEM((16, 16), jnp.int32),
            pltpu.VMEM((16, 16), jnp.int32),
            pltpu.VMEM((16, 16), jnp.float32),
            pltpu.VMEM((16, 16), jnp.float32),
            pltpu.VMEM((R,), jnp.float32),
            pltpu.VMEM((R,), jnp.float32),
            pltpu.VMEM((16, D), jnp.float32),
            pltpu.VMEM((16,), jnp.int32),
            pltpu.VMEM_SHARED((R, D), jnp.float32),
            pltpu.SemaphoreType.DMA,
        ],
        compiler_params=_sc_params(),
    )
    return kern(cnts, srcl, rll, w0l, w1l, asum, xh)


def _mlp_body(agg_ref, bg_ref, wa_ref, ba_ref, w1_ref, b1_ref, w2_ref, b2_ref,
              z_ref):
    h = jnp.maximum(agg_ref[...] + bg_ref[...], 0.0)
    h = jnp.dot(h, wa_ref[...], preferred_element_type=jnp.float32)
    h = jnp.maximum(h + ba_ref[...], 0.0)
    h = jnp.dot(h, w1_ref[...], preferred_element_type=jnp.float32)
    h = jnp.maximum(h + b1_ref[...], 0.0)
    z = jnp.dot(h, w2_ref[...], preferred_element_type=jnp.float32)
    z_ref[...] = z + b2_ref[...]


def _stage_mlp(agg, bg, wa, ba, w1, b1, w2, b2):
    return pl.pallas_call(
        _mlp_body,
        grid=(N // ROW_T,),
        in_specs=[
            pl.BlockSpec((ROW_T, D), lambda i: (i, 0)),
            pl.BlockSpec((1, D), lambda i: (0, 0)),
            pl.BlockSpec((D, 128), lambda i: (0, 0)),
            pl.BlockSpec((1, 128), lambda i: (0, 0)),
            pl.BlockSpec((128, 64), lambda i: (0, 0)),
            pl.BlockSpec((1, 64), lambda i: (0, 0)),
            pl.BlockSpec((64, 3), lambda i: (0, 0)),
            pl.BlockSpec((1, 3), lambda i: (0, 0)),
        ],
        out_specs=pl.BlockSpec((ROW_T, 3), lambda i: (i, 0)),
        out_shape=jax.ShapeDtypeStruct((N, 3), jnp.float32),
    )(agg, bg, wa, ba, w1, b1, w2, b2)


def _cdist_body(z_ref, zt_ref, out_ref):
    zi = z_ref[...]
    zt = zt_ref[...]
    sqi = jnp.sum(zi * zi, axis=1, keepdims=True)
    sqj = jnp.sum(zt * zt, axis=0, keepdims=True)
    mm = jnp.dot(zi, zt, preferred_element_type=jnp.float32)
    d2 = jnp.maximum(sqi + sqj - 2.0 * mm, 0.0)
    msk = d2 > 1e-12
    out_ref[...] = jnp.where(msk, jnp.sqrt(jnp.where(msk, d2, 1.0)), 0.0)


def _stage_cdist(z, zt):
    ncol = pl.cdiv(N, COL_T)
    return pl.pallas_call(
        _cdist_body,
        grid=(N // ROW_T, ncol),
        in_specs=[
            pl.BlockSpec((ROW_T, 3), lambda i, j: (i, 0)),
            pl.BlockSpec((3, COL_T), lambda i, j: (0, j)),
        ],
        out_specs=pl.BlockSpec((ROW_T, COL_T), lambda i, j: (i, j)),
        out_shape=jax.ShapeDtypeStruct((N, N), jnp.float32),
    )(z, zt)


def kernel(x, edge_index, W_gat, att_src, att_dst, b_gat, Wa, ba, W1, b1, W2,
           b2):
    atts = att_src.reshape(1, D)
    attd = att_dst.reshape(1, D)
    xh, asrc, adst, m = _stage1(x, W_gat, atts, attd)

    loops = jnp.arange(N, dtype=jnp.int32)
    zpad = jnp.zeros((EP - E_REAL,), jnp.int32)
    src_all = jnp.concatenate([edge_index[0].astype(jnp.int32), loops, zpad])
    dst_all = jnp.concatenate([edge_index[1].astype(jnp.int32), loops, zpad])
    src3 = src_all.reshape(NS, NR, IDX_ROW)
    dst3 = dst_all.reshape(NS, NR, IDX_ROW)

    as0 = asrc[:, 0]
    as1 = asrc[:, 1]
    ad0 = adst[:, 0]
    ad1 = adst[:, 1]
    m0 = jnp.broadcast_to(m[0, 0], (16,))
    m1 = jnp.broadcast_to(m[0, 1], (16,))

    cnts, srcl, rll, w0l, w1l, asum = _stage_attn(src3, dst3, as0, as1,
                                                  ad0, ad1, m0, m1)
    agg = _stage_agg(cnts, srcl, rll, w0l, w1l, asum, xh)

    z = _stage_mlp(agg, b_gat.reshape(1, D), Wa, ba.reshape(1, 128),
                   W1, b1.reshape(1, 64), W2, b2.reshape(1, 3))
    zt = z.T
    return _stage_cdist(z, zt)


# speedup vs baseline: 15.9439x; 2.8037x over previous
import dataclasses
import functools

import jax
import jax.numpy as jnp
from jax import lax
from jax.experimental import pallas as pl
from jax.experimental.pallas import tpu as pltpu
from jax.experimental.pallas import tpu_sc as plsc

N = 10000
E = 160000
D_IN = 512
H = 2
D_OUT = 512
D = H * D_OUT
E_REAL = E + N
NS = 16
IDX_ROW = 128
EP = 172032
PW = EP // NS
NR = PW // IDX_ROW
NG2 = PW // 16 + 16
R = 1280
NPASS = 4
N_PAD = 2 * R * NPASS
ROW_T = 400
COL_T = 1024


def _sc_params():
    cp = pltpu.CompilerParams()
    fields = pltpu.CompilerParams.__dataclass_fields__
    if "needs_layout_passes" in fields:
        cp = dataclasses.replace(cp, needs_layout_passes=False)
    if "use_tc_tiling_on_sc" in fields:
        cp = dataclasses.replace(cp, use_tc_tiling_on_sc=False)
    return cp


def _stage1_body(x_ref, w_ref, atts_ref, attd_ref, xh_ref, asrc_ref, adst_ref,
                 m_ref):
    i = pl.program_id(0)
    xh = jnp.dot(x_ref[...], w_ref[...], preferred_element_type=jnp.float32)
    xh_ref[...] = xh
    rs = xh * atts_ref[...]
    rd = xh * attd_ref[...]
    as0 = jnp.sum(rs[:, :D_OUT], axis=1)
    as1 = jnp.sum(rs[:, D_OUT:], axis=1)
    ad0 = jnp.sum(rd[:, :D_OUT], axis=1)
    ad1 = jnp.sum(rd[:, D_OUT:], axis=1)
    asrc_ref[...] = jnp.stack([as0, as1], axis=1)
    adst_ref[...] = jnp.stack([ad0, ad1], axis=1)
    mt = jnp.stack([jnp.max(as0), jnp.max(as1)])[None, :]

    @pl.when(i == 0)
    def _():
        m_ref[...] = mt

    @pl.when(i > 0)
    def _():
        m_ref[...] = jnp.maximum(m_ref[...], mt)


def _stage1(x, w_gat, atts, attd):
    return pl.pallas_call(
        _stage1_body,
        grid=(N // ROW_T,),
        in_specs=[
            pl.BlockSpec((ROW_T, D_IN), lambda i: (i, 0)),
            pl.BlockSpec((D_IN, D), lambda i: (0, 0)),
            pl.BlockSpec((1, D), lambda i: (0, 0)),
            pl.BlockSpec((1, D), lambda i: (0, 0)),
        ],
        out_specs=[
            pl.BlockSpec((ROW_T, D), lambda i: (i, 0)),
            pl.BlockSpec((ROW_T, 2), lambda i: (i, 0)),
            pl.BlockSpec((ROW_T, 2), lambda i: (i, 0)),
            pl.BlockSpec((1, 2), lambda i: (0, 0)),
        ],
        out_shape=[
            jax.ShapeDtypeStruct((N, D), jnp.float32),
            jax.ShapeDtypeStruct((N, 2), jnp.float32),
            jax.ShapeDtypeStruct((N, 2), jnp.float32),
            jax.ShapeDtypeStruct((1, 2), jnp.float32),
        ],
    )(x, w_gat, atts, attd)


def _attn_body(src_h, dst_h, as0_h, as1_h, ad0_h, ad1_h, m0_h, m1_h,
               cnt_h, srcl_h, rll_h, w0l_h, w1l_h, asum_h,
               src_v, dst_v, as0_v, as1_v, ad0_v, ad1_v,
               m0_v, m1_v, zb_v, wr0_v, wr1_v,
               srcm_v, rlm_v, w0m_v, w1m_v, cnt_v,
               spm0, spm1, sem):
    c = lax.axis_index("c")
    s = lax.axis_index("s")
    pltpu.sync_copy(src_h.at[s], src_v)
    pltpu.sync_copy(dst_h.at[s], dst_v)
    pltpu.sync_copy(as0_h, as0_v)
    pltpu.sync_copy(as1_h, as1_v)
    pltpu.sync_copy(ad0_h, ad0_v)
    pltpu.sync_copy(ad1_h, ad1_v)
    pltpu.sync_copy(m0_h, m0_v)
    pltpu.sync_copy(m1_h, m1_v)

    m0 = m0_v[...]
    m1 = m1_v[...]
    base = s * PW

    def edge_w(r, k):
        off = r * IDX_ROW + k * 16
        s16 = src_v[r, pl.ds(k * 16, 16)]
        d16 = dst_v[r, pl.ds(k * 16, 16)]
        a_s0 = plsc.load_gather(as0_v, [s16])
        a_s1 = plsc.load_gather(as1_v, [s16])
        a_d0 = plsc.load_gather(ad0_v, [d16])
        a_d1 = plsc.load_gather(ad1_v, [d16])
        t0 = a_s0 + a_d0
        t1 = a_s1 + a_d1
        t0 = jnp.where(t0 >= 0.0, t0, 0.2 * t0)
        t1 = jnp.where(t1 >= 0.0, t1, 0.2 * t1)
        u0 = m0 + a_d0
        u1 = m1 + a_d1
        u0 = jnp.where(u0 >= 0.0, u0, 0.2 * u0)
        u1 = jnp.where(u1 >= 0.0, u1, 0.2 * u1)
        w0 = jnp.exp(t0 - u0)
        w1 = jnp.exp(t1 - u1)
        valid = (base + off + lax.iota(jnp.int32, 16)) < E_REAL
        w0 = jnp.where(valid, w0, 0.0)
        w1 = jnp.where(valid, w1, 0.0)
        return s16, d16, w0, w1, valid

    @pl.when(c == 0)
    def _():
        @pl.loop(0, 125)
        def _(i):
            zb_v[pl.ds(i * 16, 16)] = jnp.zeros((16,), jnp.float32)

        @pl.when(s == 0)
        def _():
            for q in range(5):
                pltpu.sync_copy(zb_v, spm0.at[pl.ds(q * 2000, 2000)])

        @pl.when(s == 1)
        def _():
            for q in range(5):
                pltpu.sync_copy(zb_v, spm1.at[pl.ds(q * 2000, 2000)])

        plsc.subcore_barrier()

        @pl.loop(0, NR)
        def _(r):
            for k in range(8):
                _, _, w0, w1, _ = edge_w(r, k)
                wr0_v[pl.ds(k * 16, 16)] = w0
                wr1_v[pl.ds(k * 16, 16)] = w1
            pltpu.async_copy(wr0_v, spm0.at[dst_v.at[r]], sem,
                             add=True).wait()
            pltpu.async_copy(wr1_v, spm1.at[dst_v.at[r]], sem,
                             add=True).wait()

        plsc.subcore_barrier()

        @pl.when(s == 0)
        def _():
            pltpu.sync_copy(spm0, asum_h.at[0, pl.ds(0, N)])

        @pl.when(s == 1)
        def _():
            pltpu.sync_copy(spm1, asum_h.at[1, pl.ds(0, N)])

    cnt_vec = jnp.zeros((16,), jnp.int32)
    lane = lax.iota(jnp.int32, 16)
    for p in range(NPASS):
        cbase = (p * 2 + c) * R

        def comp_body(r, cnt):
            for k in range(8):
                s16, d16, w0, w1, valid = edge_w(r, k)
                rl = d16 - cbase
                m = (rl >= 0) & (rl < R) & valid
                csum = plsc.cumsum(m.astype(jnp.int32))
                tgt = cnt + csum - 1
                row = lax.shift_right_logical(tgt, 4)
                col = lax.bitwise_and(tgt, 15)
                plsc.store_scatter(srcm_v, [row, col], s16, mask=m)
                plsc.store_scatter(rlm_v, [row, col], rl, mask=m)
                plsc.store_scatter(w0m_v, [row, col], w0, mask=m)
                plsc.store_scatter(w1m_v, [row, col], w1, mask=m)
                cnt = cnt + csum[15]
            return cnt

        cnt = lax.fori_loop(0, NR, comp_body, jnp.int32(0))

        tgt = cnt + lax.iota(jnp.int32, 16)
        row = lax.shift_right_logical(tgt, 4)
        col = lax.bitwise_and(tgt, 15)
        zf = jnp.zeros((16,), jnp.float32)
        zi = jnp.zeros((16,), jnp.int32)
        plsc.store_scatter(w0m_v, [row, col], zf)
        plsc.store_scatter(w1m_v, [row, col], zf)
        plsc.store_scatter(rlm_v, [row, col], zi)
        plsc.store_scatter(srcm_v, [row, col], zi)
        cnt_vec = jnp.where(lane == p, cnt, cnt_vec)

        pltpu.sync_copy(srcm_v, srcl_h.at[c, s, p])
        pltpu.sync_copy(rlm_v, rll_h.at[c, s, p])
        pltpu.sync_copy(w0m_v, w0l_h.at[c, s, p])
        pltpu.sync_copy(w1m_v, w1l_h.at[c, s, p])

    cnt_v[...] = cnt_vec
    pltpu.sync_copy(cnt_v, cnt_h.at[c, s])


def _stage_attn(src3, dst3, as0, as1, ad0, ad1, m0, m1):
    mesh = plsc.VectorSubcoreMesh(core_axis_name="c", subcore_axis_name="s")
    kern = pl.kernel(
        _attn_body,
        out_type=[
            jax.ShapeDtypeStruct((2, NS, 16), jnp.int32),
            jax.ShapeDtypeStruct((2, NS, NPASS, NG2, 16), jnp.int32),
            jax.ShapeDtypeStruct((2, NS, NPASS, NG2, 16), jnp.int32),
            jax.ShapeDtypeStruct((2, NS, NPASS, NG2, 16), jnp.float32),
            jax.ShapeDtypeStruct((2, NS, NPASS, NG2, 16), jnp.float32),
            jax.ShapeDtypeStruct((2, N), jnp.float32),
        ],
        mesh=mesh,
        scratch_types=[
            pltpu.VMEM((NR, IDX_ROW), jnp.int32),
            pltpu.VMEM((NR, IDX_ROW), jnp.int32),
            pltpu.VMEM((N,), jnp.float32),
            pltpu.VMEM((N,), jnp.float32),
            pltpu.VMEM((N,), jnp.float32),
            pltpu.VMEM((N,), jnp.float32),
            pltpu.VMEM((16,), jnp.float32),
            pltpu.VMEM((16,), jnp.float32),
            pltpu.VMEM((2000,), jnp.float32),
            pltpu.VMEM((IDX_ROW,), jnp.float32),
            pltpu.VMEM((IDX_ROW,), jnp.float32),
            pltpu.VMEM((NG2, 16), jnp.int32),
            pltpu.VMEM((NG2, 16), jnp.int32),
            pltpu.VMEM((NG2, 16), jnp.float32),
            pltpu.VMEM((NG2, 16), jnp.float32),
            pltpu.VMEM((16,), jnp.int32),
            pltpu.VMEM_SHARED((N,), jnp.float32),
            pltpu.VMEM_SHARED((N,), jnp.float32),
            pltpu.SemaphoreType.DMA,
        ],
        compiler_params=_sc_params(),
    )
    return kern(src3, dst3, as0, as1, ad0, ad1, m0, m1)


def _agg_body(cnt_h, srcl_h, rll_h, w0l_h, w1l_h, xh_h, agg_h,
              srcb_v, rlb_v, w0b_v, w1b_v, bufa, bufb,
              cnt_v, chunk, semga, semgb, semsa, semsb):
    c = lax.axis_index("c")
    s = lax.axis_index("s")
    pltpu.sync_copy(cnt_h.at[c, s], cnt_v)
    lane = lax.iota(jnp.int32, 16)
    rps = R // NS

    def scale_rows(j, buf):
        @pl.loop(0, 16)
        def _(e):
            jv = jnp.full((16,), j, jnp.int32)
            ev = jnp.full((16,), e, jnp.int32)
            w0 = plsc.load_gather(w0b_v, [jv, ev])
            w1 = plsc.load_gather(w1b_v, [jv, ev])
            for k in range(D_OUT // 16):
                sl = pl.ds(k * 16, 16)
                buf[e, sl] = buf[e, sl] * w0
            for k in range(D_OUT // 16, D // 16):
                sl = pl.ds(k * 16, 16)
                buf[e, sl] = buf[e, sl] * w1

    for p in range(NPASS):
        cbase = (p * 2 + c) * R

        @pl.loop(0, 16)
        def _(j):
            for k in range(D // 16):
                bufa[j, pl.ds(k * 16, 16)] = jnp.zeros((16,), jnp.float32)

        for q in range(rps // 16):
            pltpu.sync_copy(bufa, chunk.at[pl.ds(s * rps + q * 16, 16)])
        plsc.subcore_barrier()

        cnt = jnp.max(jnp.where(lane == p, cnt_v[...], jnp.int32(0)))
        nblk = lax.shift_right_logical(cnt, 8)
        nrem = lax.shift_right_logical(cnt - (nblk << 8) + 15, 4)

        def stage_block(b):
            pltpu.sync_copy(srcl_h.at[c, s, p, pl.ds(b * 16, 16)], srcb_v)
            pltpu.sync_copy(rll_h.at[c, s, p, pl.ds(b * 16, 16)], rlb_v)
            pltpu.sync_copy(w0l_h.at[c, s, p, pl.ds(b * 16, 16)], w0b_v)
            pltpu.sync_copy(w1l_h.at[c, s, p, pl.ds(b * 16, 16)], w1b_v)

        @pl.loop(0, nblk)
        def _(b):
            stage_block(b)

            @pl.loop(0, 16, step=2)
            def _(j):
                ga = pltpu.async_copy(xh_h.at[srcb_v.at[j]], bufa, semga)
                gb = pltpu.async_copy(xh_h.at[srcb_v.at[j + 1]], bufb, semgb)
                ga.wait()
                scale_rows(j, bufa)
                sa = pltpu.async_copy(bufa, chunk.at[rlb_v.at[j]], semsa,
                                      add=True)
                gb.wait()
                scale_rows(j + 1, bufb)
                sb = pltpu.async_copy(bufb, chunk.at[rlb_v.at[j + 1]], semsb,
                                      add=True)
                sa.wait()
                sb.wait()

        stage_block(nblk)

        @pl.loop(0, nrem)
        def _(j):
            pltpu.async_copy(xh_h.at[srcb_v.at[j]], bufa, semga).wait()
            scale_rows(j, bufa)
            pltpu.async_copy(bufa, chunk.at[rlb_v.at[j]], semsa,
                             add=True).wait()

        plsc.subcore_barrier()

        for q in range(rps // 16):
            r0 = s * rps + q * 16
            pltpu.sync_copy(chunk.at[pl.ds(r0, 16)],
                            agg_h.at[pl.ds(cbase + r0, 16)])
        plsc.subcore_barrier()


def _stage_agg(cnts, srcl, rll, w0l, w1l, xh):
    mesh = plsc.VectorSubcoreMesh(core_axis_name="c", subcore_axis_name="s")
    kern = pl.kernel(
        _agg_body,
        out_type=jax.ShapeDtypeStruct((N_PAD, D), jnp.float32),
        mesh=mesh,
        scratch_types=[
            pltpu.VMEM((16, 16), jnp.int32),
            pltpu.VMEM((16, 16), jnp.int32),
            pltpu.VMEM((16, 16), jnp.float32),
            pltpu.VMEM((16, 16), jnp.float32),
            pltpu.VMEM((16, D), jnp.float32),
            pltpu.VMEM((16, D), jnp.float32),
            pltpu.VMEM((16,), jnp.int32),
            pltpu.VMEM_SHARED((R, D), jnp.float32),
            pltpu.SemaphoreType.DMA,
            pltpu.SemaphoreType.DMA,
            pltpu.SemaphoreType.DMA,
            pltpu.SemaphoreType.DMA,
        ],
        compiler_params=_sc_params(),
    )
    return kern(cnts, srcl, rll, w0l, w1l, xh)


def _mlp_body(agg_ref, as0_ref, as1_ref, bg_ref, wa_ref, ba_ref, w1_ref,
              b1_ref, w2_ref, b2_ref, z_ref):
    agg = agg_ref[...]
    n0 = agg[:, :D_OUT] / (as0_ref[...] + 1e-16)
    n1 = agg[:, D_OUT:] / (as1_ref[...] + 1e-16)
    aggn = jnp.concatenate([n0, n1], axis=1)
    h = jnp.maximum(aggn + bg_ref[...], 0.0)
    h = jnp.dot(h, wa_ref[...], preferred_element_type=jnp.float32)
    h = jnp.maximum(h + ba_ref[...], 0.0)
    h = jnp.dot(h, w1_ref[...], preferred_element_type=jnp.float32)
    h = jnp.maximum(h + b1_ref[...], 0.0)
    z = jnp.dot(h, w2_ref[...], preferred_element_type=jnp.float32)
    z_ref[...] = z + b2_ref[...]


def _stage_mlp(agg, as0, as1, bg, wa, ba, w1, b1, w2, b2):
    return pl.pallas_call(
        _mlp_body,
        grid=(N // ROW_T,),
        in_specs=[
            pl.BlockSpec((ROW_T, D), lambda i: (i, 0)),
            pl.BlockSpec((ROW_T, 1), lambda i: (i, 0)),
            pl.BlockSpec((ROW_T, 1), lambda i: (i, 0)),
            pl.BlockSpec((1, D), lambda i: (0, 0)),
            pl.BlockSpec((D, 128), lambda i: (0, 0)),
            pl.BlockSpec((1, 128), lambda i: (0, 0)),
            pl.BlockSpec((128, 64), lambda i: (0, 0)),
            pl.BlockSpec((1, 64), lambda i: (0, 0)),
            pl.BlockSpec((64, 3), lambda i: (0, 0)),
            pl.BlockSpec((1, 3), lambda i: (0, 0)),
        ],
        out_specs=pl.BlockSpec((ROW_T, 3), lambda i: (i, 0)),
        out_shape=jax.ShapeDtypeStruct((N, 3), jnp.float32),
    )(agg, as0, as1, bg, wa, ba, w1, b1, w2, b2)


def _cdist_body(z_ref, zt_ref, out_ref):
    zi = z_ref[...]
    zt = zt_ref[...]
    sqi = jnp.sum(zi * zi, axis=1, keepdims=True)
    sqj = jnp.sum(zt * zt, axis=0, keepdims=True)
    mm = jnp.dot(zi, zt, preferred_element_type=jnp.float32)
    d2 = jnp.maximum(sqi + sqj - 2.0 * mm, 0.0)
    msk = d2 > 1e-12
    out_ref[...] = jnp.where(msk, jnp.sqrt(jnp.where(msk, d2, 1.0)), 0.0)


def _stage_cdist(z, zt):
    ncol = pl.cdiv(N, COL_T)
    return pl.pallas_call(
        _cdist_body,
        grid=(N // ROW_T, ncol),
        in_specs=[
            pl.BlockSpec((ROW_T, 3), lambda i, j: (i, 0)),
            pl.BlockSpec((3, COL_T), lambda i, j: (0, j)),
        ],
        out_specs=pl.BlockSpec((ROW_T, COL_T), lambda i, j: (i, j)),
        out_shape=jax.ShapeDtypeStruct((N, N), jnp.float32),
    )(z, zt)


def kernel(x, edge_index, W_gat, att_src, att_dst, b_gat, Wa, ba, W1, b1, W2,
           b2):
    atts = att_src.reshape(1, D)
    attd = att_dst.reshape(1, D)
    xh, asrc, adst, m = _stage1(x, W_gat, atts, attd)

    loops = jnp.arange(N, dtype=jnp.int32)
    zpad = jnp.zeros((EP - E_REAL,), jnp.int32)
    src_all = jnp.concatenate([edge_index[0].astype(jnp.int32), loops, zpad])
    dst_all = jnp.concatenate([edge_index[1].astype(jnp.int32), loops, zpad])
    src3 = src_all.reshape(NS, NR, IDX_ROW)
    dst3 = dst_all.reshape(NS, NR, IDX_ROW)

    as0 = asrc[:, 0]
    as1 = asrc[:, 1]
    ad0 = adst[:, 0]
    ad1 = adst[:, 1]
    m0 = jnp.broadcast_to(m[0, 0], (16,))
    m1 = jnp.broadcast_to(m[0, 1], (16,))

    cnts, srcl, rll, w0l, w1l, asum = _stage_attn(src3, dst3, as0, as1,
                                                  ad0, ad1, m0, m1)
    agg = _stage_agg(cnts, srcl, rll, w0l, w1l, xh)

    z = _stage_mlp(agg, asum[0, :N].reshape(N, 1), asum[1, :N].reshape(N, 1),
                   b_gat.reshape(1, D), Wa, ba.reshape(1, 128),
                   W1, b1.reshape(1, 64), W2, b2.reshape(1, 3))
    zt = z.T
    return _stage_cdist(z, zt)


# speedup vs baseline: 16.0731x; 1.0081x over previous
import dataclasses
import functools

import jax
import jax.numpy as jnp
from jax import lax
from jax.experimental import pallas as pl
from jax.experimental.pallas import tpu as pltpu
from jax.experimental.pallas import tpu_sc as plsc

N = 10000
E = 160000
D_IN = 512
H = 2
D_OUT = 512
D = H * D_OUT
E_REAL = E + N
NS = 16
IDX_ROW = 128
EP = 172032
PW = EP // NS
NR = PW // IDX_ROW
NG2 = PW // 16 + 16
R = 1024
NPASS = 5
N_PAD = 2 * R * NPASS
ROW_T = 400
COL_T = 1024


def _sc_params():
    cp = pltpu.CompilerParams()
    fields = pltpu.CompilerParams.__dataclass_fields__
    if "needs_layout_passes" in fields:
        cp = dataclasses.replace(cp, needs_layout_passes=False)
    if "use_tc_tiling_on_sc" in fields:
        cp = dataclasses.replace(cp, use_tc_tiling_on_sc=False)
    return cp


def _stage1_body(x_ref, w_ref, atts_ref, attd_ref, xh_ref, asrc_ref, adst_ref,
                 m_ref):
    i = pl.program_id(0)
    xh = jnp.dot(x_ref[...], w_ref[...], preferred_element_type=jnp.float32)
    xh_ref[...] = xh
    rs = xh * atts_ref[...]
    rd = xh * attd_ref[...]
    as0 = jnp.sum(rs[:, :D_OUT], axis=1)
    as1 = jnp.sum(rs[:, D_OUT:], axis=1)
    ad0 = jnp.sum(rd[:, :D_OUT], axis=1)
    ad1 = jnp.sum(rd[:, D_OUT:], axis=1)
    asrc_ref[...] = jnp.stack([as0, as1], axis=1)
    adst_ref[...] = jnp.stack([ad0, ad1], axis=1)
    mt = jnp.stack([jnp.max(as0), jnp.max(as1)])[None, :]

    @pl.when(i == 0)
    def _():
        m_ref[...] = mt

    @pl.when(i > 0)
    def _():
        m_ref[...] = jnp.maximum(m_ref[...], mt)


def _stage1(x, w_gat, atts, attd):
    return pl.pallas_call(
        _stage1_body,
        grid=(N // ROW_T,),
        in_specs=[
            pl.BlockSpec((ROW_T, D_IN), lambda i: (i, 0)),
            pl.BlockSpec((D_IN, D), lambda i: (0, 0)),
            pl.BlockSpec((1, D), lambda i: (0, 0)),
            pl.BlockSpec((1, D), lambda i: (0, 0)),
        ],
        out_specs=[
            pl.BlockSpec((ROW_T, D), lambda i: (i, 0)),
            pl.BlockSpec((ROW_T, 2), lambda i: (i, 0)),
            pl.BlockSpec((ROW_T, 2), lambda i: (i, 0)),
            pl.BlockSpec((1, 2), lambda i: (0, 0)),
        ],
        out_shape=[
            jax.ShapeDtypeStruct((N, D), jnp.float32),
            jax.ShapeDtypeStruct((N, 2), jnp.float32),
            jax.ShapeDtypeStruct((N, 2), jnp.float32),
            jax.ShapeDtypeStruct((1, 2), jnp.float32),
        ],
    )(x, w_gat, atts, attd)


def _attn_body(src_h, dst_h, as0_h, as1_h, ad0_h, ad1_h, m0_h, m1_h,
               cnt_h, srcl_h, rll_h, w0l_h, w1l_h, asum_h,
               src_v, dst_v, as0_v, as1_v, ad0_v, ad1_v,
               m0_v, m1_v, zb_v, wr0_v, wr1_v,
               srcm_v, rlm_v, w0m_v, w1m_v, cnt_v,
               spm0, spm1, sem):
    c = lax.axis_index("c")
    s = lax.axis_index("s")
    pltpu.sync_copy(src_h.at[s], src_v)
    pltpu.sync_copy(dst_h.at[s], dst_v)
    pltpu.sync_copy(as0_h, as0_v)
    pltpu.sync_copy(as1_h, as1_v)
    pltpu.sync_copy(ad0_h, ad0_v)
    pltpu.sync_copy(ad1_h, ad1_v)
    pltpu.sync_copy(m0_h, m0_v)
    pltpu.sync_copy(m1_h, m1_v)

    m0 = m0_v[...]
    m1 = m1_v[...]
    base = s * PW

    def edge_w(r, k):
        off = r * IDX_ROW + k * 16
        s16 = src_v[r, pl.ds(k * 16, 16)]
        d16 = dst_v[r, pl.ds(k * 16, 16)]
        a_s0 = plsc.load_gather(as0_v, [s16])
        a_s1 = plsc.load_gather(as1_v, [s16])
        a_d0 = plsc.load_gather(ad0_v, [d16])
        a_d1 = plsc.load_gather(ad1_v, [d16])
        t0 = a_s0 + a_d0
        t1 = a_s1 + a_d1
        t0 = jnp.where(t0 >= 0.0, t0, 0.2 * t0)
        t1 = jnp.where(t1 >= 0.0, t1, 0.2 * t1)
        u0 = m0 + a_d0
        u1 = m1 + a_d1
        u0 = jnp.where(u0 >= 0.0, u0, 0.2 * u0)
        u1 = jnp.where(u1 >= 0.0, u1, 0.2 * u1)
        w0 = jnp.exp(t0 - u0)
        w1 = jnp.exp(t1 - u1)
        valid = (base + off + lax.iota(jnp.int32, 16)) < E_REAL
        w0 = jnp.where(valid, w0, 0.0)
        w1 = jnp.where(valid, w1, 0.0)
        return s16, d16, w0, w1, valid

    @pl.when(c == 0)
    def _():
        @pl.loop(0, 125)
        def _(i):
            zb_v[pl.ds(i * 16, 16)] = jnp.zeros((16,), jnp.float32)

        @pl.when(s == 0)
        def _():
            for q in range(5):
                pltpu.sync_copy(zb_v, spm0.at[pl.ds(q * 2000, 2000)])

        @pl.when(s == 1)
        def _():
            for q in range(5):
                pltpu.sync_copy(zb_v, spm1.at[pl.ds(q * 2000, 2000)])

        plsc.subcore_barrier()

        @pl.loop(0, NR)
        def _(r):
            for k in range(8):
                _, _, w0, w1, _ = edge_w(r, k)
                wr0_v[pl.ds(k * 16, 16)] = w0
                wr1_v[pl.ds(k * 16, 16)] = w1
            pltpu.async_copy(wr0_v, spm0.at[dst_v.at[r]], sem,
                             add=True).wait()
            pltpu.async_copy(wr1_v, spm1.at[dst_v.at[r]], sem,
                             add=True).wait()

        plsc.subcore_barrier()

        @pl.when(s == 0)
        def _():
            pltpu.sync_copy(spm0, asum_h.at[0, pl.ds(0, N)])

        @pl.when(s == 1)
        def _():
            pltpu.sync_copy(spm1, asum_h.at[1, pl.ds(0, N)])

    cnt_vec = jnp.zeros((16,), jnp.int32)
    lane = lax.iota(jnp.int32, 16)
    for p in range(NPASS):
        cbase = (p * 2 + c) * R

        def comp_body(r, cnt):
            for k in range(8):
                s16, d16, w0, w1, valid = edge_w(r, k)
                rl = d16 - cbase
                m = (rl >= 0) & (rl < R) & valid
                csum = plsc.cumsum(m.astype(jnp.int32))
                tgt = cnt + csum - 1
                row = lax.shift_right_logical(tgt, 4)
                col = lax.bitwise_and(tgt, 15)
                plsc.store_scatter(srcm_v, [row, col], s16, mask=m)
                plsc.store_scatter(rlm_v, [row, col], rl, mask=m)
                plsc.store_scatter(w0m_v, [row, col], w0, mask=m)
                plsc.store_scatter(w1m_v, [row, col], w1, mask=m)
                cnt = cnt + csum[15]
            return cnt

        cnt = lax.fori_loop(0, NR, comp_body, jnp.int32(0))

        zf = jnp.zeros((16,), jnp.float32)
        zi = jnp.zeros((16,), jnp.int32)
        for g in range(3):
            tgt = cnt + g * 16 + lax.iota(jnp.int32, 16)
            row = lax.shift_right_logical(tgt, 4)
            col = lax.bitwise_and(tgt, 15)
            plsc.store_scatter(w0m_v, [row, col], zf)
            plsc.store_scatter(w1m_v, [row, col], zf)
            plsc.store_scatter(rlm_v, [row, col], zi)
            plsc.store_scatter(srcm_v, [row, col], zi)
        cnt_vec = jnp.where(lane == p, cnt, cnt_vec)

        pltpu.sync_copy(srcm_v, srcl_h.at[c, s, p])
        pltpu.sync_copy(rlm_v, rll_h.at[c, s, p])
        pltpu.sync_copy(w0m_v, w0l_h.at[c, s, p])
        pltpu.sync_copy(w1m_v, w1l_h.at[c, s, p])

    cnt_v[...] = cnt_vec
    pltpu.sync_copy(cnt_v, cnt_h.at[c, s])


def _stage_attn(src3, dst3, as0, as1, ad0, ad1, m0, m1):
    mesh = plsc.VectorSubcoreMesh(core_axis_name="c", subcore_axis_name="s")
    kern = pl.kernel(
        _attn_body,
        out_type=[
            jax.ShapeDtypeStruct((2, NS, 16), jnp.int32),
            jax.ShapeDtypeStruct((2, NS, NPASS, NG2, 16), jnp.int32),
            jax.ShapeDtypeStruct((2, NS, NPASS, NG2, 16), jnp.int32),
            jax.ShapeDtypeStruct((2, NS, NPASS, NG2, 16), jnp.float32),
            jax.ShapeDtypeStruct((2, NS, NPASS, NG2, 16), jnp.float32),
            jax.ShapeDtypeStruct((2, N), jnp.float32),
        ],
        mesh=mesh,
        scratch_types=[
            pltpu.VMEM((NR, IDX_ROW), jnp.int32),
            pltpu.VMEM((NR, IDX_ROW), jnp.int32),
            pltpu.VMEM((N,), jnp.float32),
            pltpu.VMEM((N,), jnp.float32),
            pltpu.VMEM((N,), jnp.float32),
            pltpu.VMEM((N,), jnp.float32),
            pltpu.VMEM((16,), jnp.float32),
            pltpu.VMEM((16,), jnp.float32),
            pltpu.VMEM((2000,), jnp.float32),
            pltpu.VMEM((IDX_ROW,), jnp.float32),
            pltpu.VMEM((IDX_ROW,), jnp.float32),
            pltpu.VMEM((NG2, 16), jnp.int32),
            pltpu.VMEM((NG2, 16), jnp.int32),
            pltpu.VMEM((NG2, 16), jnp.float32),
            pltpu.VMEM((NG2, 16), jnp.float32),
            pltpu.VMEM((16,), jnp.int32),
            pltpu.VMEM_SHARED((N,), jnp.float32),
            pltpu.VMEM_SHARED((N,), jnp.float32),
            pltpu.SemaphoreType.DMA,
        ],
        compiler_params=_sc_params(),
    )
    return kern(src3, dst3, as0, as1, ad0, ad1, m0, m1)


def _agg_body(cnt_h, srcl_h, rll_h, w0l_h, w1l_h, xh_h, agg_h,
              srcb_v, rlb_v, w0b_v, w1b_v, bufa, bufb,
              cnt_v, chunk, semga, semgb, semsa, semsb):
    c = lax.axis_index("c")
    s = lax.axis_index("s")
    pltpu.sync_copy(cnt_h.at[c, s], cnt_v)
    lane = lax.iota(jnp.int32, 16)
    rps = R // NS

    def scale_rows(j, buf):
        @pl.loop(0, 16)
        def _(e):
            jv = jnp.full((16,), j, jnp.int32)
            ev = jnp.full((16,), e, jnp.int32)
            w0 = plsc.load_gather(w0b_v, [jv, ev])
            w1 = plsc.load_gather(w1b_v, [jv, ev])
            for k in range(D_OUT // 16):
                sl = pl.ds(k * 16, 16)
                buf[e, sl] = buf[e, sl] * w0
            for k in range(D_OUT // 16, D // 16):
                sl = pl.ds(k * 16, 16)
                buf[e, sl] = buf[e, sl] * w1

    for p in range(NPASS):
        cbase = (p * 2 + c) * R

        @pl.loop(0, 16)
        def _(j):
            for k in range(D // 16):
                bufa[j, pl.ds(k * 16, 16)] = jnp.zeros((16,), jnp.float32)

        for q in range(rps // 16):
            pltpu.sync_copy(bufa, chunk.at[pl.ds(s * rps + q * 16, 16)])
        plsc.subcore_barrier()

        cnt = jnp.max(jnp.where(lane == p, cnt_v[...], jnp.int32(0)))
        nblk = lax.shift_right_logical(cnt, 8)
        nrem = lax.shift_right_logical(cnt - (nblk << 8) + 15, 4)

        def stage_block(b):
            pltpu.sync_copy(srcl_h.at[c, s, p, pl.ds(b * 16, 16)], srcb_v)
            pltpu.sync_copy(rll_h.at[c, s, p, pl.ds(b * 16, 16)], rlb_v)
            pltpu.sync_copy(w0l_h.at[c, s, p, pl.ds(b * 16, 16)], w0b_v)
            pltpu.sync_copy(w1l_h.at[c, s, p, pl.ds(b * 16, 16)], w1b_v)

        def do_pair(j):
            ga = pltpu.async_copy(xh_h.at[srcb_v.at[j]], bufa, semga)
            gb = pltpu.async_copy(xh_h.at[srcb_v.at[j + 1]], bufb, semgb)
            ga.wait()
            scale_rows(j, bufa)
            sa = pltpu.async_copy(bufa, chunk.at[rlb_v.at[j]], semsa,
                                  add=True)
            gb.wait()
            scale_rows(j + 1, bufb)
            sb = pltpu.async_copy(bufb, chunk.at[rlb_v.at[j + 1]], semsb,
                                  add=True)
            sa.wait()
            sb.wait()

        @pl.loop(0, nblk)
        def _(b):
            stage_block(b)

            @pl.loop(0, 16, step=2)
            def _(j):
                do_pair(j)

        stage_block(nblk)

        r2 = lax.bitwise_and(nrem + 1, ~1)

        @pl.loop(0, r2, step=2)
        def _(j):
            do_pair(j)

        plsc.subcore_barrier()

        for q in range(rps // 16):
            r0 = s * rps + q * 16
            pltpu.sync_copy(chunk.at[pl.ds(r0, 16)],
                            agg_h.at[pl.ds(cbase + r0, 16)])
        plsc.subcore_barrier()


def _stage_agg(cnts, srcl, rll, w0l, w1l, xh):
    mesh = plsc.VectorSubcoreMesh(core_axis_name="c", subcore_axis_name="s")
    kern = pl.kernel(
        _agg_body,
        out_type=jax.ShapeDtypeStruct((N_PAD, D), jnp.float32),
        mesh=mesh,
        scratch_types=[
            pltpu.VMEM((16, 16), jnp.int32),
            pltpu.VMEM((16, 16), jnp.int32),
            pltpu.VMEM((16, 16), jnp.float32),
            pltpu.VMEM((16, 16), jnp.float32),
            pltpu.VMEM((16, D), jnp.float32),
            pltpu.VMEM((16, D), jnp.float32),
            pltpu.VMEM((16,), jnp.int32),
            pltpu.VMEM_SHARED((R, D), jnp.float32),
            pltpu.SemaphoreType.DMA,
            pltpu.SemaphoreType.DMA,
            pltpu.SemaphoreType.DMA,
            pltpu.SemaphoreType.DMA,
        ],
        compiler_params=_sc_params(),
    )
    return kern(cnts, srcl, rll, w0l, w1l, xh)


def _mlp_body(agg_ref, as0_ref, as1_ref, bg_ref, wa_ref, ba_ref, w1_ref,
              b1_ref, w2_ref, b2_ref, z_ref):
    agg = agg_ref[...]
    n0 = agg[:, :D_OUT] / (as0_ref[...] + 1e-16)
    n1 = agg[:, D_OUT:] / (as1_ref[...] + 1e-16)
    aggn = jnp.concatenate([n0, n1], axis=1)
    h = jnp.maximum(aggn + bg_ref[...], 0.0)
    h = jnp.dot(h, wa_ref[...], preferred_element_type=jnp.float32)
    h = jnp.maximum(h + ba_ref[...], 0.0)
    h = jnp.dot(h, w1_ref[...], preferred_element_type=jnp.float32)
    h = jnp.maximum(h + b1_ref[...], 0.0)
    z = jnp.dot(h, w2_ref[...], preferred_element_type=jnp.float32)
    z_ref[...] = z + b2_ref[...]


def _stage_mlp(agg, as0, as1, bg, wa, ba, w1, b1, w2, b2):
    return pl.pallas_call(
        _mlp_body,
        grid=(N // ROW_T,),
        in_specs=[
            pl.BlockSpec((ROW_T, D), lambda i: (i, 0)),
            pl.BlockSpec((ROW_T, 1), lambda i: (i, 0)),
            pl.BlockSpec((ROW_T, 1), lambda i: (i, 0)),
            pl.BlockSpec((1, D), lambda i: (0, 0)),
            pl.BlockSpec((D, 128), lambda i: (0, 0)),
            pl.BlockSpec((1, 128), lambda i: (0, 0)),
            pl.BlockSpec((128, 64), lambda i: (0, 0)),
            pl.BlockSpec((1, 64), lambda i: (0, 0)),
            pl.BlockSpec((64, 3), lambda i: (0, 0)),
            pl.BlockSpec((1, 3), lambda i: (0, 0)),
        ],
        out_specs=pl.BlockSpec((ROW_T, 3), lambda i: (i, 0)),
        out_shape=jax.ShapeDtypeStruct((N, 3), jnp.float32),
    )(agg, as0, as1, bg, wa, ba, w1, b1, w2, b2)


def _cdist_body(z_ref, zt_ref, out_ref):
    zi = z_ref[...]
    zt = zt_ref[...]
    sqi = jnp.sum(zi * zi, axis=1, keepdims=True)
    sqj = jnp.sum(zt * zt, axis=0, keepdims=True)
    mm = jnp.dot(zi, zt, preferred_element_type=jnp.float32)
    d2 = jnp.maximum(sqi + sqj - 2.0 * mm, 0.0)
    msk = d2 > 1e-12
    out_ref[...] = jnp.where(msk, jnp.sqrt(jnp.where(msk, d2, 1.0)), 0.0)


def _stage_cdist(z, zt):
    ncol = pl.cdiv(N, COL_T)
    return pl.pallas_call(
        _cdist_body,
        grid=(N // ROW_T, ncol),
        in_specs=[
            pl.BlockSpec((ROW_T, 3), lambda i, j: (i, 0)),
            pl.BlockSpec((3, COL_T), lambda i, j: (0, j)),
        ],
        out_specs=pl.BlockSpec((ROW_T, COL_T), lambda i, j: (i, j)),
        out_shape=jax.ShapeDtypeStruct((N, N), jnp.float32),
    )(z, zt)


def kernel(x, edge_index, W_gat, att_src, att_dst, b_gat, Wa, ba, W1, b1, W2,
           b2):
    atts = att_src.reshape(1, D)
    attd = att_dst.reshape(1, D)
    xh, asrc, adst, m = _stage1(x, W_gat, atts, attd)

    loops = jnp.arange(N, dtype=jnp.int32)
    zpad = jnp.zeros((EP - E_REAL,), jnp.int32)
    src_all = jnp.concatenate([edge_index[0].astype(jnp.int32), loops, zpad])
    dst_all = jnp.concatenate([edge_index[1].astype(jnp.int32), loops, zpad])
    src3 = src_all.reshape(NS, NR, IDX_ROW)
    dst3 = dst_all.reshape(NS, NR, IDX_ROW)

    as0 = asrc[:, 0]
    as1 = asrc[:, 1]
    ad0 = adst[:, 0]
    ad1 = adst[:, 1]
    m0 = jnp.broadcast_to(m[0, 0], (16,))
    m1 = jnp.broadcast_to(m[0, 1], (16,))

    cnts, srcl, rll, w0l, w1l, asum = _stage_attn(src3, dst3, as0, as1,
                                                  ad0, ad1, m0, m1)
    agg = _stage_agg(cnts, srcl, rll, w0l, w1l, xh)

    z = _stage_mlp(agg, asum[0, :N].reshape(N, 1), asum[1, :N].reshape(N, 1),
                   b_gat.reshape(1, D), Wa, ba.reshape(1, 128),
                   W1, b1.reshape(1, 64), W2, b2.reshape(1, 3))
    zt = z.T
    return _stage_cdist(z, zt)


# speedup vs baseline: 16.0989x; 1.0016x over previous
import dataclasses
import functools

import jax
import jax.numpy as jnp
from jax import lax
from jax.experimental import pallas as pl
from jax.experimental.pallas import tpu as pltpu
from jax.experimental.pallas import tpu_sc as plsc

N = 10000
E = 160000
D_IN = 512
H = 2
D_OUT = 512
D = H * D_OUT
E_REAL = E + N
NS = 16
IDX_ROW = 128
EP = 172032
PW = EP // NS
NR = PW // IDX_ROW
NG2 = PW // 16 + 16
R = 1024
NPASS = 5
N_PAD = 2 * R * NPASS
ROW_T = 400
COL_T = 1024


def _sc_params():
    cp = pltpu.CompilerParams()
    fields = pltpu.CompilerParams.__dataclass_fields__
    if "needs_layout_passes" in fields:
        cp = dataclasses.replace(cp, needs_layout_passes=False)
    if "use_tc_tiling_on_sc" in fields:
        cp = dataclasses.replace(cp, use_tc_tiling_on_sc=False)
    return cp


def _stage1_body(x_ref, w_ref, atts_ref, attd_ref, xh_ref, asrc_ref, adst_ref,
                 m_ref):
    i = pl.program_id(0)
    xh = jnp.dot(x_ref[...], w_ref[...], preferred_element_type=jnp.float32)
    xh_ref[...] = xh
    rs = xh * atts_ref[...]
    rd = xh * attd_ref[...]
    as0 = jnp.sum(rs[:, :D_OUT], axis=1)
    as1 = jnp.sum(rs[:, D_OUT:], axis=1)
    ad0 = jnp.sum(rd[:, :D_OUT], axis=1)
    ad1 = jnp.sum(rd[:, D_OUT:], axis=1)
    asrc_ref[...] = jnp.stack([as0, as1], axis=1)
    adst_ref[...] = jnp.stack([ad0, ad1], axis=1)
    mt = jnp.stack([jnp.max(as0), jnp.max(as1)])[None, :]

    @pl.when(i == 0)
    def _():
        m_ref[...] = mt

    @pl.when(i > 0)
    def _():
        m_ref[...] = jnp.maximum(m_ref[...], mt)


def _stage1(x, w_gat, atts, attd):
    return pl.pallas_call(
        _stage1_body,
        grid=(N // ROW_T,),
        in_specs=[
            pl.BlockSpec((ROW_T, D_IN), lambda i: (i, 0)),
            pl.BlockSpec((D_IN, D), lambda i: (0, 0)),
            pl.BlockSpec((1, D), lambda i: (0, 0)),
            pl.BlockSpec((1, D), lambda i: (0, 0)),
        ],
        out_specs=[
            pl.BlockSpec((ROW_T, D), lambda i: (i, 0)),
            pl.BlockSpec((ROW_T, 2), lambda i: (i, 0)),
            pl.BlockSpec((ROW_T, 2), lambda i: (i, 0)),
            pl.BlockSpec((1, 2), lambda i: (0, 0)),
        ],
        out_shape=[
            jax.ShapeDtypeStruct((N, D), jnp.float32),
            jax.ShapeDtypeStruct((N, 2), jnp.float32),
            jax.ShapeDtypeStruct((N, 2), jnp.float32),
            jax.ShapeDtypeStruct((1, 2), jnp.float32),
        ],
    )(x, w_gat, atts, attd)


def _attn_body(src_h, dst_h, as0_h, as1_h, ad0_h, ad1_h, m0_h, m1_h,
               cnt_h, srcl_h, rll_h, w0l_h, w1l_h, asum_h,
               src_v, dst_v, as0_v, as1_v, ad0_v, ad1_v,
               m0_v, m1_v, zb_v, wr0_v, wr1_v,
               srcm_v, rlm_v, w0m_v, w1m_v, cnt_v,
               spm0, spm1, sem):
    c = lax.axis_index("c")
    s = lax.axis_index("s")
    pltpu.sync_copy(src_h.at[s], src_v)
    pltpu.sync_copy(dst_h.at[s], dst_v)
    pltpu.sync_copy(as0_h, as0_v)
    pltpu.sync_copy(as1_h, as1_v)
    pltpu.sync_copy(ad0_h, ad0_v)
    pltpu.sync_copy(ad1_h, ad1_v)
    pltpu.sync_copy(m0_h, m0_v)
    pltpu.sync_copy(m1_h, m1_v)

    m0 = m0_v[...]
    m1 = m1_v[...]
    base = s * PW

    def edge_w(r, k):
        off = r * IDX_ROW + k * 16
        s16 = src_v[r, pl.ds(k * 16, 16)]
        d16 = dst_v[r, pl.ds(k * 16, 16)]
        a_s0 = plsc.load_gather(as0_v, [s16])
        a_s1 = plsc.load_gather(as1_v, [s16])
        a_d0 = plsc.load_gather(ad0_v, [d16])
        a_d1 = plsc.load_gather(ad1_v, [d16])
        t0 = a_s0 + a_d0
        t1 = a_s1 + a_d1
        t0 = jnp.where(t0 >= 0.0, t0, 0.2 * t0)
        t1 = jnp.where(t1 >= 0.0, t1, 0.2 * t1)
        u0 = m0 + a_d0
        u1 = m1 + a_d1
        u0 = jnp.where(u0 >= 0.0, u0, 0.2 * u0)
        u1 = jnp.where(u1 >= 0.0, u1, 0.2 * u1)
        w0 = jnp.exp(t0 - u0)
        w1 = jnp.exp(t1 - u1)
        valid = (base + off + lax.iota(jnp.int32, 16)) < E_REAL
        w0 = jnp.where(valid, w0, 0.0)
        w1 = jnp.where(valid, w1, 0.0)
        return s16, d16, w0, w1, valid

    @pl.when(c == 0)
    def _():
        @pl.loop(0, 125)
        def _(i):
            zb_v[pl.ds(i * 16, 16)] = jnp.zeros((16,), jnp.float32)

        @pl.when(s == 0)
        def _():
            for q in range(5):
                pltpu.sync_copy(zb_v, spm0.at[pl.ds(q * 2000, 2000)])

        @pl.when(s == 1)
        def _():
            for q in range(5):
                pltpu.sync_copy(zb_v, spm1.at[pl.ds(q * 2000, 2000)])

        plsc.subcore_barrier()

        @pl.loop(0, NR)
        def _(r):
            for k in range(8):
                _, _, w0, w1, _ = edge_w(r, k)
                wr0_v[pl.ds(k * 16, 16)] = w0
                wr1_v[pl.ds(k * 16, 16)] = w1
            pltpu.async_copy(wr0_v, spm0.at[dst_v.at[r]], sem,
                             add=True).wait()
            pltpu.async_copy(wr1_v, spm1.at[dst_v.at[r]], sem,
                             add=True).wait()

        plsc.subcore_barrier()

        @pl.when(s == 0)
        def _():
            pltpu.sync_copy(spm0, asum_h.at[0, pl.ds(0, N)])

        @pl.when(s == 1)
        def _():
            pltpu.sync_copy(spm1, asum_h.at[1, pl.ds(0, N)])

    cnt_vec = jnp.zeros((16,), jnp.int32)
    lane = lax.iota(jnp.int32, 16)
    for p in range(NPASS):
        cbase = (p * 2 + c) * R

        def comp_body(r, cnt):
            for k in range(8):
                s16, d16, w0, w1, valid = edge_w(r, k)
                rl = d16 - cbase
                m = (rl >= 0) & (rl < R) & valid
                csum = plsc.cumsum(m.astype(jnp.int32))
                tgt = cnt + csum - 1
                row = lax.shift_right_logical(tgt, 4)
                col = lax.bitwise_and(tgt, 15)
                plsc.store_scatter(srcm_v, [row, col], s16, mask=m)
                plsc.store_scatter(rlm_v, [row, col], rl, mask=m)
                plsc.store_scatter(w0m_v, [row, col], w0, mask=m)
                plsc.store_scatter(w1m_v, [row, col], w1, mask=m)
                cnt = cnt + csum[15]
            return cnt

        cnt = lax.fori_loop(0, NR, comp_body, jnp.int32(0))

        zf = jnp.zeros((16,), jnp.float32)
        zi = jnp.zeros((16,), jnp.int32)
        for g in range(3):
            tgt = cnt + g * 16 + lax.iota(jnp.int32, 16)
            row = lax.shift_right_logical(tgt, 4)
            col = lax.bitwise_and(tgt, 15)
            plsc.store_scatter(w0m_v, [row, col], zf)
            plsc.store_scatter(w1m_v, [row, col], zf)
            plsc.store_scatter(rlm_v, [row, col], zi)
            plsc.store_scatter(srcm_v, [row, col], zi)
        cnt_vec = jnp.where(lane == p, cnt, cnt_vec)

        pltpu.sync_copy(srcm_v, srcl_h.at[c, s, p])
        pltpu.sync_copy(rlm_v, rll_h.at[c, s, p])
        pltpu.sync_copy(w0m_v, w0l_h.at[c, s, p])
        pltpu.sync_copy(w1m_v, w1l_h.at[c, s, p])

    cnt_v[...] = cnt_vec
    pltpu.sync_copy(cnt_v, cnt_h.at[c, s])


def _stage_attn(src3, dst3, as0, as1, ad0, ad1, m0, m1):
    mesh = plsc.VectorSubcoreMesh(core_axis_name="c", subcore_axis_name="s")
    kern = pl.kernel(
        _attn_body,
        out_type=[
            jax.ShapeDtypeStruct((2, NS, 16), jnp.int32),
            jax.ShapeDtypeStruct((2, NS, NPASS, NG2, 16), jnp.int32),
            jax.ShapeDtypeStruct((2, NS, NPASS, NG2, 16), jnp.int32),
            jax.ShapeDtypeStruct((2, NS, NPASS, NG2, 16), jnp.float32),
            jax.ShapeDtypeStruct((2, NS, NPASS, NG2, 16), jnp.float32),
            jax.ShapeDtypeStruct((2, N), jnp.float32),
        ],
        mesh=mesh,
        scratch_types=[
            pltpu.VMEM((NR, IDX_ROW), jnp.int32),
            pltpu.VMEM((NR, IDX_ROW), jnp.int32),
            pltpu.VMEM((N,), jnp.float32),
            pltpu.VMEM((N,), jnp.float32),
            pltpu.VMEM((N,), jnp.float32),
            pltpu.VMEM((N,), jnp.float32),
            pltpu.VMEM((16,), jnp.float32),
            pltpu.VMEM((16,), jnp.float32),
            pltpu.VMEM((2000,), jnp.float32),
            pltpu.VMEM((IDX_ROW,), jnp.float32),
            pltpu.VMEM((IDX_ROW,), jnp.float32),
            pltpu.VMEM((NG2, 16), jnp.int32),
            pltpu.VMEM((NG2, 16), jnp.int32),
            pltpu.VMEM((NG2, 16), jnp.float32),
            pltpu.VMEM((NG2, 16), jnp.float32),
            pltpu.VMEM((16,), jnp.int32),
            pltpu.VMEM_SHARED((N,), jnp.float32),
            pltpu.VMEM_SHARED((N,), jnp.float32),
            pltpu.SemaphoreType.DMA,
        ],
        compiler_params=_sc_params(),
    )
    return kern(src3, dst3, as0, as1, ad0, ad1, m0, m1)


def _agg_body(cnt_h, srcl_h, rll_h, w0l_h, w1l_h, xh_h, agg_h,
              srcb_v, rlb_v, w0b_v, w1b_v, bufa, bufb,
              cnt_v, chunk, semga, semgb, semsa, semsb):
    c = lax.axis_index("c")
    s = lax.axis_index("s")
    pltpu.sync_copy(cnt_h.at[c, s], cnt_v)
    lane = lax.iota(jnp.int32, 16)
    rps = R // NS

    def scale_rows(j, buf):
        @pl.loop(0, 16, step=2)
        def _(e):
            jv = jnp.full((16,), j, jnp.int32)
            ev = jnp.full((16,), e, jnp.int32)
            fv = ev + 1
            w0 = plsc.load_gather(w0b_v, [jv, ev])
            w1 = plsc.load_gather(w1b_v, [jv, ev])
            u0 = plsc.load_gather(w0b_v, [jv, fv])
            u1 = plsc.load_gather(w1b_v, [jv, fv])
            f = e + 1
            for k in range(D_OUT // 16):
                sl = pl.ds(k * 16, 16)
                buf[e, sl] = buf[e, sl] * w0
                buf[f, sl] = buf[f, sl] * u0
            for k in range(D_OUT // 16, D // 16):
                sl = pl.ds(k * 16, 16)
                buf[e, sl] = buf[e, sl] * w1
                buf[f, sl] = buf[f, sl] * u1

    for p in range(NPASS):
        cbase = (p * 2 + c) * R

        @pl.loop(0, 16)
        def _(j):
            for k in range(D // 16):
                bufa[j, pl.ds(k * 16, 16)] = jnp.zeros((16,), jnp.float32)

        for q in range(rps // 16):
            pltpu.sync_copy(bufa, chunk.at[pl.ds(s * rps + q * 16, 16)])
        plsc.subcore_barrier()

        cnt = jnp.max(jnp.where(lane == p, cnt_v[...], jnp.int32(0)))
        nblk = lax.shift_right_logical(cnt, 8)
        nrem = lax.shift_right_logical(cnt - (nblk << 8) + 15, 4)

        def stage_block(b):
            pltpu.sync_copy(srcl_h.at[c, s, p, pl.ds(b * 16, 16)], srcb_v)
            pltpu.sync_copy(rll_h.at[c, s, p, pl.ds(b * 16, 16)], rlb_v)
            pltpu.sync_copy(w0l_h.at[c, s, p, pl.ds(b * 16, 16)], w0b_v)
            pltpu.sync_copy(w1l_h.at[c, s, p, pl.ds(b * 16, 16)], w1b_v)

        def do_pair(j):
            ga = pltpu.async_copy(xh_h.at[srcb_v.at[j]], bufa, semga)
            gb = pltpu.async_copy(xh_h.at[srcb_v.at[j + 1]], bufb, semgb)
            ga.wait()
            scale_rows(j, bufa)
            sa = pltpu.async_copy(bufa, chunk.at[rlb_v.at[j]], semsa,
                                  add=True)
            gb.wait()
            scale_rows(j + 1, bufb)
            sb = pltpu.async_copy(bufb, chunk.at[rlb_v.at[j + 1]], semsb,
                                  add=True)
            sa.wait()
            sb.wait()

        @pl.loop(0, nblk)
        def _(b):
            stage_block(b)

            @pl.loop(0, 16, step=2)
            def _(j):
                do_pair(j)

        stage_block(nblk)

        r2 = lax.bitwise_and(nrem + 1, ~1)

        @pl.loop(0, r2, step=2)
        def _(j):
            do_pair(j)

        plsc.subcore_barrier()

        for q in range(rps // 16):
            r0 = s * rps + q * 16
            pltpu.sync_copy(chunk.at[pl.ds(r0, 16)],
                            agg_h.at[pl.ds(cbase + r0, 16)])
        plsc.subcore_barrier()


def _stage_agg(cnts, srcl, rll, w0l, w1l, xh):
    mesh = plsc.VectorSubcoreMesh(core_axis_name="c", subcore_axis_name="s")
    kern = pl.kernel(
        _agg_body,
        out_type=jax.ShapeDtypeStruct((N_PAD, D), jnp.float32),
        mesh=mesh,
        scratch_types=[
            pltpu.VMEM((16, 16), jnp.int32),
            pltpu.VMEM((16, 16), jnp.int32),
            pltpu.VMEM((16, 16), jnp.float32),
            pltpu.VMEM((16, 16), jnp.float32),
            pltpu.VMEM((16, D), jnp.float32),
            pltpu.VMEM((16, D), jnp.float32),
            pltpu.VMEM((16,), jnp.int32),
            pltpu.VMEM_SHARED((R, D), jnp.float32),
            pltpu.SemaphoreType.DMA,
            pltpu.SemaphoreType.DMA,
            pltpu.SemaphoreType.DMA,
            pltpu.SemaphoreType.DMA,
        ],
        compiler_params=_sc_params(),
    )
    return kern(cnts, srcl, rll, w0l, w1l, xh)


def _mlp_body(agg_ref, as0_ref, as1_ref, bg_ref, wa_ref, ba_ref, w1_ref,
              b1_ref, w2_ref, b2_ref, z_ref):
    agg = agg_ref[...]
    n0 = agg[:, :D_OUT] / (as0_ref[...] + 1e-16)
    n1 = agg[:, D_OUT:] / (as1_ref[...] + 1e-16)
    aggn = jnp.concatenate([n0, n1], axis=1)
    h = jnp.maximum(aggn + bg_ref[...], 0.0)
    h = jnp.dot(h, wa_ref[...], preferred_element_type=jnp.float32)
    h = jnp.maximum(h + ba_ref[...], 0.0)
    h = jnp.dot(h, w1_ref[...], preferred_element_type=jnp.float32)
    h = jnp.maximum(h + b1_ref[...], 0.0)
    z = jnp.dot(h, w2_ref[...], preferred_element_type=jnp.float32)
    z_ref[...] = z + b2_ref[...]


def _stage_mlp(agg, as0, as1, bg, wa, ba, w1, b1, w2, b2):
    return pl.pallas_call(
        _mlp_body,
        grid=(N // ROW_T,),
        in_specs=[
            pl.BlockSpec((ROW_T, D), lambda i: (i, 0)),
            pl.BlockSpec((ROW_T, 1), lambda i: (i, 0)),
            pl.BlockSpec((ROW_T, 1), lambda i: (i, 0)),
            pl.BlockSpec((1, D), lambda i: (0, 0)),
            pl.BlockSpec((D, 128), lambda i: (0, 0)),
            pl.BlockSpec((1, 128), lambda i: (0, 0)),
            pl.BlockSpec((128, 64), lambda i: (0, 0)),
            pl.BlockSpec((1, 64), lambda i: (0, 0)),
            pl.BlockSpec((64, 3), lambda i: (0, 0)),
            pl.BlockSpec((1, 3), lambda i: (0, 0)),
        ],
        out_specs=pl.BlockSpec((ROW_T, 3), lambda i: (i, 0)),
        out_shape=jax.ShapeDtypeStruct((N, 3), jnp.float32),
    )(agg, as0, as1, bg, wa, ba, w1, b1, w2, b2)


def _cdist_body(z_ref, zt_ref, out_ref):
    zi = z_ref[...]
    zt = zt_ref[...]
    sqi = jnp.sum(zi * zi, axis=1, keepdims=True)
    sqj = jnp.sum(zt * zt, axis=0, keepdims=True)
    mm = jnp.dot(zi, zt, preferred_element_type=jnp.float32)
    d2 = jnp.maximum(sqi + sqj - 2.0 * mm, 0.0)
    msk = d2 > 1e-12
    out_ref[...] = jnp.where(msk, jnp.sqrt(jnp.where(msk, d2, 1.0)), 0.0)


def _stage_cdist(z, zt):
    ncol = pl.cdiv(N, COL_T)
    return pl.pallas_call(
        _cdist_body,
        grid=(N // ROW_T, ncol),
        in_specs=[
            pl.BlockSpec((ROW_T, 3), lambda i, j: (i, 0)),
            pl.BlockSpec((3, COL_T), lambda i, j: (0, j)),
        ],
        out_specs=pl.BlockSpec((ROW_T, COL_T), lambda i, j: (i, j)),
        out_shape=jax.ShapeDtypeStruct((N, N), jnp.float32),
    )(z, zt)


def kernel(x, edge_index, W_gat, att_src, att_dst, b_gat, Wa, ba, W1, b1, W2,
           b2):
    atts = att_src.reshape(1, D)
    attd = att_dst.reshape(1, D)
    xh, asrc, adst, m = _stage1(x, W_gat, atts, attd)

    loops = jnp.arange(N, dtype=jnp.int32)
    zpad = jnp.zeros((EP - E_REAL,), jnp.int32)
    src_all = jnp.concatenate([edge_index[0].astype(jnp.int32), loops, zpad])
    dst_all = jnp.concatenate([edge_index[1].astype(jnp.int32), loops, zpad])
    src3 = src_all.reshape(NS, NR, IDX_ROW)
    dst3 = dst_all.reshape(NS, NR, IDX_ROW)

    as0 = asrc[:, 0]
    as1 = asrc[:, 1]
    ad0 = adst[:, 0]
    ad1 = adst[:, 1]
    m0 = jnp.broadcast_to(m[0, 0], (16,))
    m1 = jnp.broadcast_to(m[0, 1], (16,))

    cnts, srcl, rll, w0l, w1l, asum = _stage_attn(src3, dst3, as0, as1,
                                                  ad0, ad1, m0, m1)
    agg = _stage_agg(cnts, srcl, rll, w0l, w1l, xh)

    z = _stage_mlp(agg, asum[0, :N].reshape(N, 1), asum[1, :N].reshape(N, 1),
                   b_gat.reshape(1, D), Wa, ba.reshape(1, 128),
                   W1, b1.reshape(1, 64), W2, b2.reshape(1, 3))
    zt = z.T
    return _stage_cdist(z, zt)
